# baseline (speedup 1.0000x reference)
import functools

import jax
import jax.numpy as jnp
from jax.experimental import pallas as pl
from jax.experimental.pallas import tpu as pltpu

NUM_FILTERS = 2
N = 2048
D = 128
EPS = 1e-5


def _layer_norm(x, g, b):
    m = jnp.mean(x, axis=-1, keepdims=True)
    v = jnp.mean((x - m) ** 2, axis=-1, keepdims=True)
    return (x - m) / jnp.sqrt(v + EPS) * g + b


def _body(x_ref, f_ref, w1_ref, b1_ref, g1_ref, be1_ref,
          w2_ref, b2_ref, g2_ref, be2_ref,
          w3_ref, b3_ref, g3_ref, be3_ref, o_ref):
    x = x_ref[...]
    f = f_ref[...]

    fb = f.astype(jnp.bfloat16)

    def conv_layer(h, w_ref, b_ref):
        c = jnp.dot(fb, h.astype(jnp.bfloat16),
                    preferred_element_type=jnp.float32)
        w = w_ref[...]
        z = (jnp.dot(c[:N], w[:D], preferred_element_type=jnp.float32)
             + jnp.dot(c[N:], w[D:], preferred_element_type=jnp.float32)
             + b_ref[...])
        return jax.nn.relu(z)

    h = conv_layer(x, w1_ref, b1_ref)
    h = _layer_norm(h, g1_ref[...], be1_ref[...])
    h = conv_layer(h, w2_ref, b2_ref)
    h = _layer_norm(h, g2_ref[...], be2_ref[...])
    h = conv_layer(h, w3_ref, b3_ref)
    out = _layer_norm(x + h, g3_ref[...], be3_ref[...])
    o_ref[...] = jax.nn.relu(out)


@functools.partial(jax.jit)
def _run(X, filt, W1, b1, g1, be1, W2, b2, g2, be2, W3, b3, g3, be3):
    x2 = X.reshape(N, D)
    f2 = filt.reshape(NUM_FILTERS * N, N)
    vecs = [v.reshape(1, D) for v in (b1, g1, be1, b2, g2, be2, b3, g3, be3)]
    b1r, g1r, be1r, b2r, g2r, be2r, b3r, g3r, be3r = vecs
    out = pl.pallas_call(
        _body,
        out_shape=jax.ShapeDtypeStruct((N, D), jnp.float32),
        compiler_params=pltpu.CompilerParams(
            vmem_limit_bytes=100 * 1024 * 1024,
        ),
    )(x2, f2, W1, b1r, g1r, be1r, W2, b2r, g2r, be2r, W3, b3r, g3r, be3r)
    return out.reshape(1, N, D)


def kernel(X, graph_conv_filters_input, W1, b1, g1, be1,
           W2, b2, g2, be2, W3, b3, g3, be3):
    return _run(X, graph_conv_filters_input, W1, b1, g1, be1,
                W2, b2, g2, be2, W3, b3, g3, be3)


# speedup vs baseline: 1.0540x; 1.0540x over previous
import functools

import jax
import jax.numpy as jnp
from jax.experimental import pallas as pl
from jax.experimental.pallas import tpu as pltpu

NUM_FILTERS = 2
N = 2048
D = 128
EPS = 1e-5

GRID = 8
BLK = (NUM_FILTERS * N) // GRID


def _layer_norm(x, g, b):
    m = jnp.mean(x, axis=-1, keepdims=True)
    v = jnp.mean((x - m) ** 2, axis=-1, keepdims=True)
    return (x - m) / jnp.sqrt(v + EPS) * g + b


def _body(x_ref, fblk_ref, w1_ref, b1_ref, g1_ref, be1_ref,
          w2_ref, b2_ref, g2_ref, be2_ref,
          w3_ref, b3_ref, g3_ref, be3_ref, o_ref,
          fb_scr, c_scr):
    i = pl.program_id(0)

    fb = fblk_ref[...].astype(jnp.bfloat16)
    fb_scr[pl.ds(i * BLK, BLK), :] = fb
    xb = x_ref[...].astype(jnp.bfloat16)
    c_scr[pl.ds(i * BLK, BLK), :] = jnp.dot(
        fb, xb, preferred_element_type=jnp.float32)

    @pl.when(i == GRID - 1)
    def _tail():
        def dense_relu(c, w_ref, b_ref):
            w = w_ref[...].astype(jnp.bfloat16)
            cb = c.astype(jnp.bfloat16)
            z = (jnp.dot(cb[:N], w[:D], preferred_element_type=jnp.float32)
                 + jnp.dot(cb[N:], w[D:], preferred_element_type=jnp.float32)
                 + b_ref[...])
            return jax.nn.relu(z)

        def conv_layer(h, w_ref, b_ref):
            c = jnp.dot(fb_scr[...], h.astype(jnp.bfloat16),
                        preferred_element_type=jnp.float32)
            return dense_relu(c, w_ref, b_ref)

        h = dense_relu(c_scr[...], w1_ref, b1_ref)
        h = _layer_norm(h, g1_ref[...], be1_ref[...])
        h = conv_layer(h, w2_ref, b2_ref)
        h = _layer_norm(h, g2_ref[...], be2_ref[...])
        h = conv_layer(h, w3_ref, b3_ref)
        out = _layer_norm(x_ref[...] + h, g3_ref[...], be3_ref[...])
        o_ref[...] = jax.nn.relu(out)


@functools.partial(jax.jit)
def _run(X, filt, W1, b1, g1, be1, W2, b2, g2, be2, W3, b3, g3, be3):
    x2 = X.reshape(N, D)
    f2 = filt.reshape(NUM_FILTERS * N, N)
    vecs = [v.reshape(1, D) for v in (b1, g1, be1, b2, g2, be2, b3, g3, be3)]
    b1r, g1r, be1r, b2r, g2r, be2r, b3r, g3r, be3r = vecs
    const = lambda i: (0, 0)
    out = pl.pallas_call(
        _body,
        grid=(GRID,),
        in_specs=[
            pl.BlockSpec((N, D), const),
            pl.BlockSpec((BLK, N), lambda i: (i, 0)),
            pl.BlockSpec((NUM_FILTERS * D, D), const),
            pl.BlockSpec((1, D), const),
            pl.BlockSpec((1, D), const),
            pl.BlockSpec((1, D), const),
            pl.BlockSpec((NUM_FILTERS * D, D), const),
            pl.BlockSpec((1, D), const),
            pl.BlockSpec((1, D), const),
            pl.BlockSpec((1, D), const),
            pl.BlockSpec((NUM_FILTERS * D, D), const),
            pl.BlockSpec((1, D), const),
            pl.BlockSpec((1, D), const),
            pl.BlockSpec((1, D), const),
        ],
        out_specs=pl.BlockSpec((N, D), const),
        out_shape=jax.ShapeDtypeStruct((N, D), jnp.float32),
        scratch_shapes=[
            pltpu.VMEM((NUM_FILTERS * N, N), jnp.bfloat16),
            pltpu.VMEM((NUM_FILTERS * N, D), jnp.float32),
        ],
        compiler_params=pltpu.CompilerParams(
            dimension_semantics=("arbitrary",),
            vmem_limit_bytes=100 * 1024 * 1024,
        ),
    )(x2, f2, W1, b1r, g1r, be1r, W2, b2r, g2r, be2r, W3, b3r, g3r, be3r)
    return out.reshape(1, N, D)


def kernel(X, graph_conv_filters_input, W1, b1, g1, be1,
           W2, b2, g2, be2, W3, b3, g3, be3):
    return _run(X, graph_conv_filters_input, W1, b1, g1, be1,
                W2, b2, g2, be2, W3, b3, g3, be3)


# speedup vs baseline: 1.0588x; 1.0046x over previous
import functools

import jax
import jax.numpy as jnp
from jax.experimental import pallas as pl
from jax.experimental.pallas import tpu as pltpu

NUM_FILTERS = 2
N = 2048
D = 128
EPS = 1e-5

NBLK = 8
BLK = (NUM_FILTERS * N) // NBLK
NSTAGE = 4


def _layer_norm(x, g, b):
    m = jnp.mean(x, axis=-1, keepdims=True)
    v = jnp.mean((x - m) ** 2, axis=-1, keepdims=True)
    return (x - m) / jnp.sqrt(v + EPS) * g + b


def _body(x_ref, f_hbm, w1_ref, b1_ref, g1_ref, be1_ref,
          w2_ref, b2_ref, g2_ref, be2_ref,
          w3_ref, b3_ref, g3_ref, be3_ref, o_ref,
          stage, fb_scr, c_scr, sems):

    def copy(i):
        return pltpu.make_async_copy(
            f_hbm.at[pl.ds(i * BLK, BLK), :],
            stage.at[i % NSTAGE],
            sems.at[i],
        )

    for i in range(NSTAGE):
        copy(i).start()

    xb = x_ref[...].astype(jnp.bfloat16)

    for i in range(NBLK):
        copy(i).wait()
        fb = stage[i % NSTAGE].astype(jnp.bfloat16)
        if i + NSTAGE < NBLK:
            copy(i + NSTAGE).start()
        fb_scr[pl.ds(i * BLK, BLK), :] = fb
        c_scr[pl.ds(i * BLK, BLK), :] = jnp.dot(
            fb, xb, preferred_element_type=jnp.float32)

    def dense_relu(c, w_ref, b_ref):
        w = w_ref[...].astype(jnp.bfloat16)
        cb = c.astype(jnp.bfloat16)
        z = (jnp.dot(cb[:N], w[:D], preferred_element_type=jnp.float32)
             + jnp.dot(cb[N:], w[D:], preferred_element_type=jnp.float32)
             + b_ref[...])
        return jax.nn.relu(z)

    def conv_layer(h, w_ref, b_ref):
        c = jnp.dot(fb_scr[...], h.astype(jnp.bfloat16),
                    preferred_element_type=jnp.float32)
        return dense_relu(c, w_ref, b_ref)

    h = dense_relu(c_scr[...], w1_ref, b1_ref)
    h = _layer_norm(h, g1_ref[...], be1_ref[...])
    h = conv_layer(h, w2_ref, b2_ref)
    h = _layer_norm(h, g2_ref[...], be2_ref[...])
    h = conv_layer(h, w3_ref, b3_ref)
    out = _layer_norm(x_ref[...] + h, g3_ref[...], be3_ref[...])
    o_ref[...] = jax.nn.relu(out)


@functools.partial(jax.jit)
def _run(X, filt, W1, b1, g1, be1, W2, b2, g2, be2, W3, b3, g3, be3):
    x2 = X.reshape(N, D)
    f2 = filt.reshape(NUM_FILTERS * N, N)
    vecs = [v.reshape(1, D) for v in (b1, g1, be1, b2, g2, be2, b3, g3, be3)]
    b1r, g1r, be1r, b2r, g2r, be2r, b3r, g3r, be3r = vecs
    vspec = pl.BlockSpec(memory_space=pltpu.MemorySpace.VMEM)
    out = pl.pallas_call(
        _body,
        in_specs=[
            vspec,
            pl.BlockSpec(memory_space=pltpu.MemorySpace.HBM),
            vspec, vspec, vspec, vspec,
            vspec, vspec, vspec, vspec,
            vspec, vspec, vspec, vspec,
        ],
        out_specs=vspec,
        out_shape=jax.ShapeDtypeStruct((N, D), jnp.float32),
        scratch_shapes=[
            pltpu.VMEM((NSTAGE, BLK, N), jnp.float32),
            pltpu.VMEM((NUM_FILTERS * N, N), jnp.bfloat16),
            pltpu.VMEM((NUM_FILTERS * N, D), jnp.float32),
            pltpu.SemaphoreType.DMA((NBLK,)),
        ],
        compiler_params=pltpu.CompilerParams(
            vmem_limit_bytes=100 * 1024 * 1024,
        ),
    )(x2, f2, W1, b1r, g1r, be1r, W2, b2r, g2r, be2r, W3, b3r, g3r, be3r)
    return out.reshape(1, N, D)


def kernel(X, graph_conv_filters_input, W1, b1, g1, be1,
           W2, b2, g2, be2, W3, b3, g3, be3):
    return _run(X, graph_conv_filters_input, W1, b1, g1, be1,
                W2, b2, g2, be2, W3, b3, g3, be3)


# speedup vs baseline: 1.4022x; 1.3243x over previous
import functools

import jax
import jax.numpy as jnp
from jax.experimental import pallas as pl
from jax.experimental.pallas import tpu as pltpu

NUM_FILTERS = 2
N = 2048
D = 128
EPS = 1e-5

NBLK = 8
BLK = (NUM_FILTERS * N) // NBLK
HALF = NBLK // 2
NSTAGE = 4


def _layer_norm(x, g, b):
    m = jnp.mean(x, axis=-1, keepdims=True)
    v = jnp.mean((x - m) ** 2, axis=-1, keepdims=True)
    return (x - m) / jnp.sqrt(v + EPS) * g + b


def _body(x_ref, f_hbm, w1_ref, b1_ref, g1_ref, be1_ref,
          w2_ref, b2_ref, g2_ref, be2_ref,
          w3_ref, b3_ref, g3_ref, be3_ref, o_ref,
          stage, fb_scr, cc_scr, sems):

    def copy(i):
        return pltpu.make_async_copy(
            f_hbm.at[pl.ds(i * BLK, BLK), :],
            stage.at[i % NSTAGE],
            sems.at[i],
        )

    for i in range(NSTAGE):
        copy(i).start()

    xb = x_ref[...].astype(jnp.bfloat16)

    for i in range(NBLK):
        copy(i).wait()
        fb = stage[i % NSTAGE].astype(jnp.bfloat16)
        if i + NSTAGE < NBLK:
            copy(i + NSTAGE).start()
        fb_scr[pl.ds(i * BLK, BLK), :] = fb
        part = jnp.dot(fb, xb, preferred_element_type=jnp.float32)
        f, r = divmod(i, HALF)
        cc_scr[pl.ds(r * BLK, BLK), pl.ds(f * D, D)] = part.astype(jnp.bfloat16)

    def dense_relu(w_ref, b_ref):
        z = jnp.dot(cc_scr[...], w_ref[...].astype(jnp.bfloat16),
                    preferred_element_type=jnp.float32) + b_ref[...]
        return jax.nn.relu(z)

    def conv_layer(h, w_ref, b_ref):
        hb = h.astype(jnp.bfloat16)
        for f in range(NUM_FILTERS):
            part = jnp.dot(fb_scr[pl.ds(f * N, N), :], hb,
                           preferred_element_type=jnp.float32)
            cc_scr[:, pl.ds(f * D, D)] = part.astype(jnp.bfloat16)
        return dense_relu(w_ref, b_ref)

    h = dense_relu(w1_ref, b1_ref)
    h = _layer_norm(h, g1_ref[...], be1_ref[...])
    h = conv_layer(h, w2_ref, b2_ref)
    h = _layer_norm(h, g2_ref[...], be2_ref[...])
    h = conv_layer(h, w3_ref, b3_ref)
    out = _layer_norm(x_ref[...] + h, g3_ref[...], be3_ref[...])
    o_ref[...] = jax.nn.relu(out)


@functools.partial(jax.jit)
def _run(X, filt, W1, b1, g1, be1, W2, b2, g2, be2, W3, b3, g3, be3):
    x2 = X.reshape(N, D)
    f2 = filt.reshape(NUM_FILTERS * N, N)
    vecs = [v.reshape(1, D) for v in (b1, g1, be1, b2, g2, be2, b3, g3, be3)]
    b1r, g1r, be1r, b2r, g2r, be2r, b3r, g3r, be3r = vecs
    vspec = pl.BlockSpec(memory_space=pltpu.MemorySpace.VMEM)
    out = pl.pallas_call(
        _body,
        in_specs=[
            vspec,
            pl.BlockSpec(memory_space=pltpu.MemorySpace.HBM),
            vspec, vspec, vspec, vspec,
            vspec, vspec, vspec, vspec,
            vspec, vspec, vspec, vspec,
        ],
        out_specs=vspec,
        out_shape=jax.ShapeDtypeStruct((N, D), jnp.float32),
        scratch_shapes=[
            pltpu.VMEM((NSTAGE, BLK, N), jnp.float32),
            pltpu.VMEM((NUM_FILTERS * N, N), jnp.bfloat16),
            pltpu.VMEM((N, NUM_FILTERS * D), jnp.bfloat16),
            pltpu.SemaphoreType.DMA((NBLK,)),
        ],
        compiler_params=pltpu.CompilerParams(
            vmem_limit_bytes=100 * 1024 * 1024,
        ),
    )(x2, f2, W1, b1r, g1r, be1r, W2, b2r, g2r, be2r, W3, b3r, g3r, be3r)
    return out.reshape(1, N, D)


def kernel(X, graph_conv_filters_input, W1, b1, g1, be1,
           W2, b2, g2, be2, W3, b3, g3, be3):
    return _run(X, graph_conv_filters_input, W1, b1, g1, be1,
                W2, b2, g2, be2, W3, b3, g3, be3)
